# baseline (speedup 1.0000x reference)
import jax
import jax.numpy as jnp
from jax.experimental import pallas as pl


_HB = 32


def _body(x_ref, row_ref, col_ref, o_ref):
    x = x_ref[...]
    row = row_ref[...]
    col = col_ref[...]
    pos = row[:, None, :] + col[None, :, :]
    o_ref[...] = x + pos[None, :, :, :]


def kernel(inputs, row_embeddings, col_embeddings):
    b, h, w, c = inputs.shape
    hb = _HB
    grid = (b, h // hb)
    return pl.pallas_call(
        _body,
        grid=grid,
        in_specs=[
            pl.BlockSpec((1, hb, w, c), lambda bi, hi: (bi, hi, 0, 0)),
            pl.BlockSpec((hb, c), lambda bi, hi: (hi, 0)),
            pl.BlockSpec((w, c), lambda bi, hi: (0, 0)),
        ],
        out_specs=pl.BlockSpec((1, hb, w, c), lambda bi, hi: (bi, hi, 0, 0)),
        out_shape=jax.ShapeDtypeStruct((b, h, w, c), inputs.dtype),
    )(inputs, row_embeddings, col_embeddings)
